# baseline (speedup 1.0000x reference)
import jax
from jax.experimental import pallas as pl
from jax.experimental.pallas import tpu as pltpu


def _memcpy_kernel(x_ref, o_ref, sem):
    copy = pltpu.make_async_copy(x_ref, o_ref, sem)
    copy.start()
    copy.wait()


def kernel(input):
    return pl.pallas_call(
        _memcpy_kernel,
        in_specs=[pl.BlockSpec(memory_space=pl.ANY)],
        out_specs=pl.BlockSpec(memory_space=pl.ANY),
        out_shape=jax.ShapeDtypeStruct(input.shape, input.dtype),
        scratch_shapes=[pltpu.SemaphoreType.DMA],
    )(input)


# speedup vs baseline: 15.5719x; 15.5719x over previous
import jax
from jax.experimental import pallas as pl
from jax.experimental.pallas import tpu as pltpu

_ROWS = 768
_COLS = 224 * 224
_BLOCK_ROWS = 32


def _copy_block(x_ref, o_ref):
    o_ref[...] = x_ref[...]


def kernel(input):
    x = input.reshape(_ROWS, _COLS)
    out = pl.pallas_call(
        _copy_block,
        grid=(_ROWS // _BLOCK_ROWS,),
        in_specs=[pl.BlockSpec((_BLOCK_ROWS, _COLS), lambda i: (i, 0))],
        out_specs=pl.BlockSpec((_BLOCK_ROWS, _COLS), lambda i: (i, 0)),
        out_shape=jax.ShapeDtypeStruct((_ROWS, _COLS), x.dtype),
    )(x)
    return out.reshape(input.shape)


# speedup vs baseline: 15.6273x; 1.0036x over previous
import jax
from jax.experimental import pallas as pl
from jax.experimental.pallas import tpu as pltpu

_ROWS = 768
_COLS = 224 * 224
_BLOCK_ROWS = 64


def _copy_block(x_ref, o_ref):
    o_ref[...] = x_ref[...]


def kernel(input):
    x = input.reshape(_ROWS, _COLS)
    out = pl.pallas_call(
        _copy_block,
        grid=(_ROWS // _BLOCK_ROWS,),
        in_specs=[pl.BlockSpec((_BLOCK_ROWS, _COLS), lambda i: (i, 0))],
        out_specs=pl.BlockSpec((_BLOCK_ROWS, _COLS), lambda i: (i, 0)),
        out_shape=jax.ShapeDtypeStruct((_ROWS, _COLS), x.dtype),
        compiler_params=pltpu.CompilerParams(
            dimension_semantics=("parallel",),
        ),
    )(x)
    return out.reshape(input.shape)
